# Initial kernel scaffold; baseline (speedup 1.0000x reference)
import jax
import jax.numpy as jnp
from jax.experimental import pallas as pl


def kernel(x, edge_index, W1, b1, W2, b2):
    raise NotImplementedError("write your pallas kernel here")


# speedup vs baseline: 11.4578x; 11.4578x over previous
import functools

import jax
import jax.numpy as jnp
from jax import lax
from jax.experimental import pallas as pl
from jax.experimental.pallas import tpu as pltpu
from jax.experimental.pallas import tpu_sc as plsc

NN = 10000
EE = 320000
IN_C = 128
HID = 64
OUT_C = 128

NC = 2
NS = 16
NW = NC * NS

N2 = 10112
E2 = 327680
IDX_ROWS = E2 // 128
ROWS_PER_W = IDX_ROWS // NW
SUP = 8
NSUP = ROWS_PER_W // SUP
RPT = N2 // NS


def _sc_mesh():
    return plsc.VectorSubcoreMesh(core_axis_name="c", subcore_axis_name="s",
                                  num_cores=NC, num_subcores=NS)


def _sc_app_common(tbl, src2, dst2, z64, pout, acc, src_v, dst_v, rows_v,
                   gsem, ssem, z16=None, ones16=None, dout=None, dacc=None,
                   ones_v=None):
    c = lax.axis_index("c")
    s = lax.axis_index("s")
    w = s * NC + c
    r0 = s * RPT
    pltpu.sync_copy(z64.at[pl.ds(r0, RPT)], acc.at[pl.ds(r0, RPT)])
    if dacc is not None:
        pltpu.sync_copy(z16.at[pl.ds(r0, RPT)], dacc.at[pl.ds(r0, RPT)])
        pltpu.sync_copy(ones16, ones_v)
    plsc.subcore_barrier()

    base = w * ROWS_PER_W

    def chunk(i, carry):
        ro = base + i * SUP
        pltpu.sync_copy(src2.at[pl.ds(ro, SUP)], src_v)
        pltpu.sync_copy(dst2.at[pl.ds(ro, SUP)], dst_v)
        gcs = [pltpu.async_copy(tbl.at[src_v.at[j]],
                                rows_v.at[pl.ds(j * 128, 128)], gsem)
               for j in range(SUP)]
        for g in gcs:
            g.wait()
        scs = [pltpu.async_copy(rows_v.at[pl.ds(j * 128, 128)],
                                acc.at[dst_v.at[j]], ssem, add=True)
               for j in range(SUP)]
        if dacc is not None:
            scs += [pltpu.async_copy(ones_v, dacc.at[dst_v.at[j]], ssem,
                                     add=True)
                    for j in range(SUP)]
        for sc_ in scs:
            sc_.wait()
        return carry

    lax.fori_loop(0, NSUP, chunk, 0)
    plsc.subcore_barrier()
    pltpu.sync_copy(acc.at[pl.ds(r0, RPT)], pout.at[c, pl.ds(r0, RPT)])
    if dacc is not None:
        pltpu.sync_copy(dacc.at[pl.ds(r0, RPT)], dout.at[c, pl.ds(r0, RPT)])


@functools.partial(
    pl.kernel,
    out_type=(jax.ShapeDtypeStruct((NC, N2, HID), jnp.float32),
              jax.ShapeDtypeStruct((NC, N2, 16), jnp.float32)),
    mesh=_sc_mesh(),
    compiler_params=pltpu.CompilerParams(use_tc_tiling_on_sc=False),
    scratch_types=(
        pltpu.VMEM_SHARED((N2, HID), jnp.float32),
        pltpu.VMEM((SUP, 128), jnp.int32),
        pltpu.VMEM((SUP, 128), jnp.int32),
        pltpu.VMEM((SUP * 128, HID), jnp.float32),
        pltpu.SemaphoreType.DMA,
        pltpu.SemaphoreType.DMA,
        pltpu.VMEM_SHARED((N2, 16), jnp.float32),
        pltpu.VMEM((128, 16), jnp.float32),
    ),
)
def _sc_app_deg(tbl, src2, dst2, z64, z16, ones16, pout, dout, acc, src_v,
                dst_v, rows_v, gsem, ssem, dacc, ones_v):
    _sc_app_common(tbl, src2, dst2, z64, pout, acc, src_v, dst_v, rows_v,
                   gsem, ssem, z16=z16, ones16=ones16, dout=dout, dacc=dacc,
                   ones_v=ones_v)


@functools.partial(
    pl.kernel,
    out_type=jax.ShapeDtypeStruct((NC, N2, HID), jnp.float32),
    mesh=_sc_mesh(),
    compiler_params=pltpu.CompilerParams(use_tc_tiling_on_sc=False),
    scratch_types=(
        pltpu.VMEM_SHARED((N2, HID), jnp.float32),
        pltpu.VMEM((SUP, 128), jnp.int32),
        pltpu.VMEM((SUP, 128), jnp.int32),
        pltpu.VMEM((SUP * 128, HID), jnp.float32),
        pltpu.SemaphoreType.DMA,
        pltpu.SemaphoreType.DMA,
    ),
)
def _sc_app(tbl, src2, dst2, z64, pout, acc, src_v, dst_v, rows_v, gsem,
            ssem):
    _sc_app_common(tbl, src2, dst2, z64, pout, acc, src_v, dst_v, rows_v,
                   gsem, ssem)


def _zmm(x2, W1):
    def body(x_ref, w_ref, z_ref):
        for k in range(4):
            z_ref[k] = jnp.dot(x_ref[...], w_ref[k],
                               preferred_element_type=jnp.float32)

    return pl.pallas_call(
        body,
        out_shape=jax.ShapeDtypeStruct((4, N2, HID), jnp.float32),
    )(x2, W1)


def _comb1(p, degp, z):
    def body(p_ref, dp_ref, z_ref, t_ref, di_ref):
        dinv = 1.0 / jnp.maximum(dp_ref[0] + dp_ref[1], 1.0)
        di_ref[...] = dinv
        t_ref[...] = (p_ref[0] + p_ref[1]) * dinv[:, :1] + z_ref[...]

    return pl.pallas_call(
        body,
        out_shape=(jax.ShapeDtypeStruct((N2, HID), jnp.float32),
                   jax.ShapeDtypeStruct((N2, 16), jnp.float32)),
    )(p, degp, z)


def _comb_add(p, dinv, z):
    def body(p_ref, di_ref, z_ref, t_ref):
        t_ref[...] = (p_ref[0] + p_ref[1]) * di_ref[:, :1] + z_ref[...]

    return pl.pallas_call(
        body, out_shape=jax.ShapeDtypeStruct((N2, HID), jnp.float32),
    )(p, dinv, z)


def _comb_relu(p, dinv, z, b1):
    def body(p_ref, di_ref, z_ref, b_ref, t_ref):
        t = (p_ref[0] + p_ref[1]) * di_ref[:, :1] + z_ref[...] + b_ref[...]
        t_ref[...] = jnp.maximum(t, 0.0)

    return pl.pallas_call(
        body, out_shape=jax.ShapeDtypeStruct((N2, HID), jnp.float32),
    )(p, dinv, z, b1)


def _comb_plain(p, dinv):
    def body(p_ref, di_ref, t_ref):
        t_ref[...] = (p_ref[0] + p_ref[1]) * di_ref[:, :1]

    return pl.pallas_call(
        body, out_shape=jax.ShapeDtypeStruct((N2, HID), jnp.float32),
    )(p, dinv)


def _fin(h, a1, a2, a3, W2, b2):
    def body(h_ref, a1_ref, a2_ref, a3_ref, w_ref, b_ref, y_ref):
        acc = jnp.dot(h_ref[...], w_ref[0], preferred_element_type=jnp.float32)
        acc += jnp.dot(a1_ref[...], w_ref[1], preferred_element_type=jnp.float32)
        acc += jnp.dot(a2_ref[...], w_ref[2], preferred_element_type=jnp.float32)
        acc += jnp.dot(a3_ref[...], w_ref[3], preferred_element_type=jnp.float32)
        y_ref[...] = acc + b_ref[...]

    return pl.pallas_call(
        body, out_shape=jax.ShapeDtypeStruct((N2, OUT_C), jnp.float32),
    )(h, a1, a2, a3, W2, b2)


def kernel(x, edge_index, W1, b1, W2, b2):
    src = edge_index[0]
    dst = edge_index[1]
    pad = E2 - EE
    pidx = lax.iota(jnp.int32, pad)
    src2 = jnp.concatenate([src, pidx % 128]).reshape(IDX_ROWS, 128)
    dst2 = jnp.concatenate([dst, NN + (pidx % 8)]).reshape(IDX_ROWS, 128)
    x2 = jnp.pad(x, ((0, N2 - NN), (0, 0)))
    zeros64 = jnp.zeros((N2, HID), jnp.float32)
    zeros16 = jnp.zeros((N2, 16), jnp.float32)
    ones16 = jnp.ones((128, 16), jnp.float32)

    z = _zmm(x2, W1)
    p, degp = _sc_app_deg(z[3], src2, dst2, zeros64, zeros16, ones16)
    t, dinv = _comb1(p, degp, z[2])
    p = _sc_app(t, src2, dst2, zeros64)
    t = _comb_add(p, dinv, z[1])
    p = _sc_app(t, src2, dst2, zeros64)
    h = _comb_relu(p, dinv, z[0], b1)
    p = _sc_app(h, src2, dst2, zeros64)
    a1 = _comb_plain(p, dinv)
    p = _sc_app(a1, src2, dst2, zeros64)
    a2 = _comb_plain(p, dinv)
    p = _sc_app(a2, src2, dst2, zeros64)
    a3 = _comb_plain(p, dinv)
    y2 = _fin(h, a1, a2, a3, W2, b2)
    return y2[:NN]


# speedup vs baseline: 12.9745x; 1.1324x over previous
import functools

import jax
import jax.numpy as jnp
from jax import lax
from jax.experimental import pallas as pl
from jax.experimental.pallas import tpu as pltpu
from jax.experimental.pallas import tpu_sc as plsc

NN = 10000
EE = 320000
IN_C = 128
HID = 64
OUT_C = 128

NC = 2
NS = 16
NW = NC * NS

N2 = 10112
E2 = 327680
IDX_ROWS = E2 // 128
ROWS_PER_W = IDX_ROWS // NW
SUP = 8
NSUP = ROWS_PER_W // SUP
RPT = N2 // NS


def _sc_mesh():
    return plsc.VectorSubcoreMesh(core_axis_name="c", subcore_axis_name="s",
                                  num_cores=NC, num_subcores=NS)


def _sc_app_common(tbl, src2, dst2, z64, pout, acc, src_v, dst_v, rows_v,
                   gsem, ssem, z16=None, ones16=None, dout=None, dacc=None,
                   ones_v=None):
    c = lax.axis_index("c")
    s = lax.axis_index("s")
    w = s * NC + c
    r0 = s * RPT
    pltpu.sync_copy(z64.at[pl.ds(r0, RPT)], acc.at[pl.ds(r0, RPT)])
    if dacc is not None:
        pltpu.sync_copy(z16.at[pl.ds(r0, RPT)], dacc.at[pl.ds(r0, RPT)])
        pltpu.sync_copy(ones16, ones_v)
    plsc.subcore_barrier()

    base = w * ROWS_PER_W

    def _scatter_waits(j):
        pltpu.make_async_copy(tbl.at[pl.ds(0, 128)],
                              rows_v.at[pl.ds(j * 128, 128)],
                              ssem.at[j]).wait()
        if dacc is not None:
            pltpu.make_async_copy(ones16, ones_v, ssem.at[j]).wait()

    def chunk(i, carry):
        ro = base + i * SUP
        par = lax.rem(i, 2)
        pltpu.sync_copy(src2.at[pl.ds(ro, SUP)], src_v.at[par])
        pltpu.sync_copy(dst2.at[pl.ds(ro, SUP)], dst_v.at[par])

        @pl.when(i > 0)
        def _():
            for j in range(SUP):
                _scatter_waits(j)

        gcs = [pltpu.async_copy(tbl.at[src_v.at[par, j]],
                                rows_v.at[pl.ds(j * 128, 128)],
                                gsem.at[j])
               for j in range(SUP)]
        for j in range(SUP):
            gcs[j].wait()
            pltpu.async_copy(rows_v.at[pl.ds(j * 128, 128)],
                             acc.at[dst_v.at[par, j]], ssem.at[j], add=True)
            if dacc is not None:
                pltpu.async_copy(ones_v, dacc.at[dst_v.at[par, j]],
                                 ssem.at[j], add=True)
        return carry

    lax.fori_loop(0, NSUP, chunk, 0)
    for j in range(SUP):
        _scatter_waits(j)
    plsc.subcore_barrier()
    pltpu.sync_copy(acc.at[pl.ds(r0, RPT)], pout.at[c, pl.ds(r0, RPT)])
    if dacc is not None:
        pltpu.sync_copy(dacc.at[pl.ds(r0, RPT)], dout.at[c, pl.ds(r0, RPT)])


@functools.partial(
    pl.kernel,
    out_type=(jax.ShapeDtypeStruct((NC, N2, HID), jnp.float32),
              jax.ShapeDtypeStruct((NC, N2, 16), jnp.float32)),
    mesh=_sc_mesh(),
    compiler_params=pltpu.CompilerParams(use_tc_tiling_on_sc=False),
    scratch_types=(
        pltpu.VMEM_SHARED((N2, HID), jnp.float32),
        pltpu.VMEM((2, SUP, 128), jnp.int32),
        pltpu.VMEM((2, SUP, 128), jnp.int32),
        pltpu.VMEM((SUP * 128, HID), jnp.float32),
        pltpu.SemaphoreType.DMA((SUP,)),
        pltpu.SemaphoreType.DMA((SUP,)),
        pltpu.VMEM_SHARED((N2, 16), jnp.float32),
        pltpu.VMEM((128, 16), jnp.float32),
    ),
)
def _sc_app_deg(tbl, src2, dst2, z64, z16, ones16, pout, dout, acc, src_v,
                dst_v, rows_v, gsem, ssem, dacc, ones_v):
    _sc_app_common(tbl, src2, dst2, z64, pout, acc, src_v, dst_v, rows_v,
                   gsem, ssem, z16=z16, ones16=ones16, dout=dout, dacc=dacc,
                   ones_v=ones_v)


@functools.partial(
    pl.kernel,
    out_type=jax.ShapeDtypeStruct((NC, N2, HID), jnp.float32),
    mesh=_sc_mesh(),
    compiler_params=pltpu.CompilerParams(use_tc_tiling_on_sc=False),
    scratch_types=(
        pltpu.VMEM_SHARED((N2, HID), jnp.float32),
        pltpu.VMEM((2, SUP, 128), jnp.int32),
        pltpu.VMEM((2, SUP, 128), jnp.int32),
        pltpu.VMEM((SUP * 128, HID), jnp.float32),
        pltpu.SemaphoreType.DMA((SUP,)),
        pltpu.SemaphoreType.DMA((SUP,)),
    ),
)
def _sc_app(tbl, src2, dst2, z64, pout, acc, src_v, dst_v, rows_v, gsem,
            ssem):
    _sc_app_common(tbl, src2, dst2, z64, pout, acc, src_v, dst_v, rows_v,
                   gsem, ssem)


def _zmm(x2, W1):
    def body(x_ref, w_ref, z_ref):
        for k in range(4):
            z_ref[k] = jnp.dot(x_ref[...], w_ref[k],
                               preferred_element_type=jnp.float32)

    return pl.pallas_call(
        body,
        out_shape=jax.ShapeDtypeStruct((4, N2, HID), jnp.float32),
    )(x2, W1)


def _comb1(p, degp, z):
    def body(p_ref, dp_ref, z_ref, t_ref, di_ref):
        dinv = 1.0 / jnp.maximum(dp_ref[0] + dp_ref[1], 1.0)
        di_ref[...] = dinv
        t_ref[...] = (p_ref[0] + p_ref[1]) * dinv[:, :1] + z_ref[...]

    return pl.pallas_call(
        body,
        out_shape=(jax.ShapeDtypeStruct((N2, HID), jnp.float32),
                   jax.ShapeDtypeStruct((N2, 16), jnp.float32)),
    )(p, degp, z)


def _comb_add(p, dinv, z):
    def body(p_ref, di_ref, z_ref, t_ref):
        t_ref[...] = (p_ref[0] + p_ref[1]) * di_ref[:, :1] + z_ref[...]

    return pl.pallas_call(
        body, out_shape=jax.ShapeDtypeStruct((N2, HID), jnp.float32),
    )(p, dinv, z)


def _comb_relu(p, dinv, z, b1):
    def body(p_ref, di_ref, z_ref, b_ref, t_ref):
        t = (p_ref[0] + p_ref[1]) * di_ref[:, :1] + z_ref[...] + b_ref[...]
        t_ref[...] = jnp.maximum(t, 0.0)

    return pl.pallas_call(
        body, out_shape=jax.ShapeDtypeStruct((N2, HID), jnp.float32),
    )(p, dinv, z, b1)


def _comb_plain(p, dinv):
    def body(p_ref, di_ref, t_ref):
        t_ref[...] = (p_ref[0] + p_ref[1]) * di_ref[:, :1]

    return pl.pallas_call(
        body, out_shape=jax.ShapeDtypeStruct((N2, HID), jnp.float32),
    )(p, dinv)


def _fin(h, a1, a2, a3, W2, b2):
    def body(h_ref, a1_ref, a2_ref, a3_ref, w_ref, b_ref, y_ref):
        acc = jnp.dot(h_ref[...], w_ref[0], preferred_element_type=jnp.float32)
        acc += jnp.dot(a1_ref[...], w_ref[1], preferred_element_type=jnp.float32)
        acc += jnp.dot(a2_ref[...], w_ref[2], preferred_element_type=jnp.float32)
        acc += jnp.dot(a3_ref[...], w_ref[3], preferred_element_type=jnp.float32)
        y_ref[...] = acc + b_ref[...]

    return pl.pallas_call(
        body, out_shape=jax.ShapeDtypeStruct((N2, OUT_C), jnp.float32),
    )(h, a1, a2, a3, W2, b2)


def kernel(x, edge_index, W1, b1, W2, b2):
    src = edge_index[0]
    dst = edge_index[1]
    pad = E2 - EE
    pidx = lax.iota(jnp.int32, pad)
    src2 = jnp.concatenate([src, pidx % 128]).reshape(IDX_ROWS, 128)
    dst2 = jnp.concatenate([dst, NN + (pidx % 8)]).reshape(IDX_ROWS, 128)
    x2 = jnp.pad(x, ((0, N2 - NN), (0, 0)))
    zeros64 = jnp.zeros((N2, HID), jnp.float32)
    zeros16 = jnp.zeros((N2, 16), jnp.float32)
    ones16 = jnp.ones((128, 16), jnp.float32)

    z = _zmm(x2, W1)
    p, degp = _sc_app_deg(z[3], src2, dst2, zeros64, zeros16, ones16)
    t, dinv = _comb1(p, degp, z[2])
    p = _sc_app(t, src2, dst2, zeros64)
    t = _comb_add(p, dinv, z[1])
    p = _sc_app(t, src2, dst2, zeros64)
    h = _comb_relu(p, dinv, z[0], b1)
    p = _sc_app(h, src2, dst2, zeros64)
    a1 = _comb_plain(p, dinv)
    p = _sc_app(a1, src2, dst2, zeros64)
    a2 = _comb_plain(p, dinv)
    p = _sc_app(a2, src2, dst2, zeros64)
    a3 = _comb_plain(p, dinv)
    y2 = _fin(h, a1, a2, a3, W2, b2)
    return y2[:NN]
